# baseline (speedup 1.0000x reference)
import functools

import jax
import jax.numpy as jnp
from jax import lax
from jax.experimental import pallas as pl
from jax.experimental.pallas import tpu as pltpu
from jax.experimental.pallas import tpu_sc as plsc

B = 64
S = 512
H = 768
NV = H // 16
EPS = 1e-12
INV_H = 1.0 / H

NW = 32
SPW = S // NW
NCHUNK = 16
BPC = 8
PPC = 8
RPC = BPC * PPC


def _rsqrt16(v):
    i = lax.bitcast_convert_type(v, jnp.int32)
    y = lax.bitcast_convert_type(jnp.int32(0x5F3759DF) - (i >> 1), jnp.float32)
    for _ in range(3):
        y = y * (1.5 - 0.5 * v * y * y)
    return y


def _body(ids_r, tok, pos, typ, gam, bet, out,
          idx_v, pos_v, typ_v, gam_v, bet_v, rows_a, rows_b,
          gsem_a, gsem_b, ssem_a, ssem_b):
    wid = lax.axis_index("s") * 2 + lax.axis_index("c")
    s0 = wid * SPW

    pltpu.sync_copy(ids_r.at[wid], idx_v)
    pltpu.sync_copy(pos.at[pl.ds(s0, SPW)], pos_v)
    pltpu.sync_copy(typ.at[pl.ds(0, 1)], typ_v)
    pltpu.sync_copy(gam, gam_v)
    pltpu.sync_copy(bet, bet_v)

    def fold(j, _):
        sl = pl.ds(j * 16, 16)
        tv = typ_v[0, sl]
        for t in range(SPW):
            pos_v[t, sl] = pos_v[t, sl] + tv
        return 0
    lax.fori_loop(0, NV, fold, 0)

    def gather_start(c, buf, sem):
        pltpu.async_copy(tok.at[idx_v.at[c]], buf, sem)

    def gather_wait(c, buf, sem):
        pltpu.make_async_copy(tok.at[idx_v.at[c]], buf, sem).wait()

    def scatter_start(bb, p, buf, sem):
        for br in range(BPC):
            pltpu.async_copy(buf.at[pl.ds(br * PPC, PPC)],
                             out.at[bb * BPC + br, pl.ds(s0 + p * PPC, PPC)],
                             sem)

    def scatter_wait(bb, p, buf, sem):
        for br in range(BPC):
            pltpu.make_async_copy(buf.at[pl.ds(br * PPC, PPC)],
                                  out.at[bb * BPC + br,
                                         pl.ds(s0 + p * PPC, PPC)],
                                  sem).wait()

    def compute(rows, p):
        for jp in range(PPC):
            prow = p * PPC + jp

            zeros = tuple(jnp.zeros((16,), jnp.float32) for _ in range(BPC))

            def p1(j, carry, jp=jp, prow=prow):
                sl = pl.ds(j * 16, 16)
                pv = pos_v[prow, sl]
                s_acc, q_acc = carry
                s_new, q_new = [], []
                for br in range(BPC):
                    x = rows[br * PPC + jp, sl] + pv
                    rows[br * PPC + jp, sl] = x
                    s_new.append(s_acc[br] + x)
                    q_new.append(q_acc[br] + x * x)
                return (tuple(s_new), tuple(q_new))

            s_fin, q_fin = plsc.parallel_loop(
                0, NV, unroll=2, carry=(zeros, zeros))(p1)

            mean_v, rstd_v = [], []
            for br in range(BPC):
                m = jnp.sum(s_fin[br]) * INV_H
                var = jnp.sum(q_fin[br]) * INV_H - m * m
                mean_v.append(jnp.full((16,), m, jnp.float32))
                rstd_v.append(_rsqrt16(jnp.full((16,), var + EPS, jnp.float32)))

            def p2(j, jp=jp, mean_v=mean_v, rstd_v=rstd_v):
                sl = pl.ds(j * 16, 16)
                gv = gam_v[sl]
                bv = bet_v[sl]
                for br in range(BPC):
                    x = rows[br * PPC + jp, sl]
                    rows[br * PPC + jp, sl] = \
                        (x - mean_v[br]) * rstd_v[br] * gv + bv
            plsc.parallel_loop(0, NV, unroll=2)(p2)

    gather_start(0, rows_a, gsem_a)

    def pair(i, _):
        c0 = i * 2
        gather_wait(c0, rows_a, gsem_a)

        @pl.when(i > 0)
        def _():
            scatter_wait(i - 1, 1, rows_b, ssem_b)
        gather_start(c0 + 1, rows_b, gsem_b)
        compute(rows_a, 0)
        scatter_start(i, 0, rows_a, ssem_a)

        gather_wait(c0 + 1, rows_b, gsem_b)

        @pl.when(i < NCHUNK // 2 - 1)
        def _():
            scatter_wait(i, 0, rows_a, ssem_a)
            gather_start(c0 + 2, rows_a, gsem_a)
        compute(rows_b, 1)
        scatter_start(i, 1, rows_b, ssem_b)
        return 0
    lax.fori_loop(0, NCHUNK // 2, pair, 0)

    scatter_wait(NCHUNK // 2 - 1, 0, rows_a, ssem_a)
    scatter_wait(NCHUNK // 2 - 1, 1, rows_b, ssem_b)


_emb = functools.partial(
    pl.kernel,
    mesh=plsc.VectorSubcoreMesh(core_axis_name="c", subcore_axis_name="s"),
    compiler_params=pltpu.CompilerParams(needs_layout_passes=False),
    out_type=jax.ShapeDtypeStruct((B, S, H), jnp.float32),
    scratch_types=[
        pltpu.VMEM((NCHUNK, RPC), jnp.int32),
        pltpu.VMEM((SPW, H), jnp.float32),
        pltpu.VMEM((1, H), jnp.float32),
        pltpu.VMEM((H,), jnp.float32),
        pltpu.VMEM((H,), jnp.float32),
        pltpu.VMEM((RPC, H), jnp.float32),
        pltpu.VMEM((RPC, H), jnp.float32),
        pltpu.SemaphoreType.DMA,
        pltpu.SemaphoreType.DMA,
        pltpu.SemaphoreType.DMA,
        pltpu.SemaphoreType.DMA,
    ],
)(_body)


def kernel(ids, tok_table, pos_table, type_table, gamma, beta):
    ids_r = (ids.astype(jnp.int32)
             .reshape(BPC, BPC, NW, 2, PPC)
             .transpose(2, 0, 3, 1, 4)
             .reshape(NW, NCHUNK, RPC))
    return _emb(ids_r, tok_table, pos_table, type_table, gamma, beta)


# speedup vs baseline: 1.2731x; 1.2731x over previous
import functools

import jax
import jax.numpy as jnp
from jax import lax
from jax.experimental import pallas as pl
from jax.experimental.pallas import tpu as pltpu
from jax.experimental.pallas import tpu_sc as plsc

B = 64
S = 512
H = 768
NV = H // 16
EPS = 1e-12
INV_H = 1.0 / H

NW = 32
SPW = S // NW
NCHUNK = 16
BPC = 8
PPC = 8
RPC = BPC * PPC


def _rsqrt16(v):
    i = lax.bitcast_convert_type(v, jnp.int32)
    y = lax.bitcast_convert_type(jnp.int32(0x5F3759DF) - (i >> 1), jnp.float32)
    for _ in range(3):
        y = y * (1.5 - 0.5 * v * y * y)
    return y


def _body(ids_r, tok, pos, typ, gam, bet, out,
          idx_v, pos_v, typ_v, gam_v, bet_v, rows_a, rows_b,
          gsem_a, gsem_b, ssem_a, ssem_b):
    wid = lax.axis_index("s") * 2 + lax.axis_index("c")
    s0 = wid * SPW

    pltpu.sync_copy(ids_r.at[wid], idx_v)
    pltpu.sync_copy(pos.at[pl.ds(s0, SPW)], pos_v)
    pltpu.sync_copy(typ.at[pl.ds(0, 1)], typ_v)
    pltpu.sync_copy(gam, gam_v)
    pltpu.sync_copy(bet, bet_v)

    def fold(j, _):
        sl = pl.ds(j * 16, 16)
        tv = typ_v[0, sl]
        for t in range(SPW):
            pos_v[t, sl] = pos_v[t, sl] + tv
        return 0
    lax.fori_loop(0, NV, fold, 0)

    def gather_start(c, buf, sem):
        pltpu.async_copy(tok.at[idx_v.at[c]], buf, sem)

    def gather_wait(c, buf, sem):
        pltpu.make_async_copy(tok.at[idx_v.at[c]], buf, sem).wait()

    def scatter_start(bb, p, buf, sem):
        for br in range(BPC):
            pltpu.async_copy(buf.at[pl.ds(br * PPC, PPC)],
                             out.at[bb * BPC + br, pl.ds(s0 + p * PPC, PPC)],
                             sem)

    def scatter_wait(bb, p, buf, sem):
        for br in range(BPC):
            pltpu.make_async_copy(buf.at[pl.ds(br * PPC, PPC)],
                                  out.at[bb * BPC + br,
                                         pl.ds(s0 + p * PPC, PPC)],
                                  sem).wait()

    def compute(rows, p, groups):
        for jp in groups:
            prow = p * PPC + jp

            def p1(j, carry, jp=jp, prow=prow):
                sl = pl.ds(j * 16, 16)
                pv = pos_v[prow, sl]
                s_acc, q_acc = carry
                s_new, q_new = [], []
                for br in range(BPC):
                    x = rows[br * PPC + jp, sl] + pv
                    rows[br * PPC + jp, sl] = x
                    s_new.append(s_acc[br] + x)
                    q_new.append(q_acc[br] + x * x)
                return (tuple(s_new), tuple(q_new))

            zeros = tuple(jnp.zeros((16,), jnp.float32) for _ in range(BPC))
            s_fin, q_fin = lax.fori_loop(0, NV, p1, (zeros, zeros))

            mean_v, rstd_v = [], []
            for br in range(BPC):
                m = jnp.sum(s_fin[br]) * INV_H
                var = jnp.sum(q_fin[br]) * INV_H - m * m
                mean_v.append(jnp.full((16,), m, jnp.float32))
                rstd_v.append(_rsqrt16(jnp.full((16,), var + EPS, jnp.float32)))

            def p2(j, _, jp=jp, mean_v=mean_v, rstd_v=rstd_v):
                sl = pl.ds(j * 16, 16)
                gv = gam_v[sl]
                bv = bet_v[sl]
                for br in range(BPC):
                    x = rows[br * PPC + jp, sl]
                    rows[br * PPC + jp, sl] = \
                        (x - mean_v[br]) * rstd_v[br] * gv + bv
                return 0
            lax.fori_loop(0, NV, p2, 0)

    gather_start(0, rows_a, gsem_a)
    lo = range(0, PPC // 2)
    hi = range(PPC // 2, PPC)

    def pair(i, _):
        c0 = i * 2
        gather_wait(c0, rows_a, gsem_a)
        compute(rows_a, 0, lo)

        @pl.when(i > 0)
        def _():
            scatter_wait(i - 1, 1, rows_b, ssem_b)
        gather_start(c0 + 1, rows_b, gsem_b)
        compute(rows_a, 0, hi)
        scatter_start(i, 0, rows_a, ssem_a)

        gather_wait(c0 + 1, rows_b, gsem_b)
        compute(rows_b, 1, lo)
        scatter_wait(i, 0, rows_a, ssem_a)

        @pl.when(i < NCHUNK // 2 - 1)
        def _():
            gather_start(c0 + 2, rows_a, gsem_a)
        compute(rows_b, 1, hi)
        scatter_start(i, 1, rows_b, ssem_b)
        return 0
    lax.fori_loop(0, NCHUNK // 2, pair, 0)

    scatter_wait(NCHUNK // 2 - 1, 1, rows_b, ssem_b)


_emb = functools.partial(
    pl.kernel,
    mesh=plsc.VectorSubcoreMesh(core_axis_name="c", subcore_axis_name="s"),
    compiler_params=pltpu.CompilerParams(needs_layout_passes=False),
    out_type=jax.ShapeDtypeStruct((B, S, H), jnp.float32),
    scratch_types=[
        pltpu.VMEM((NCHUNK, RPC), jnp.int32),
        pltpu.VMEM((SPW, H), jnp.float32),
        pltpu.VMEM((1, H), jnp.float32),
        pltpu.VMEM((H,), jnp.float32),
        pltpu.VMEM((H,), jnp.float32),
        pltpu.VMEM((RPC, H), jnp.float32),
        pltpu.VMEM((RPC, H), jnp.float32),
        pltpu.SemaphoreType.DMA,
        pltpu.SemaphoreType.DMA,
        pltpu.SemaphoreType.DMA,
        pltpu.SemaphoreType.DMA,
    ],
)(_body)


def kernel(ids, tok_table, pos_table, type_table, gamma, beta):
    ids_r = (ids.astype(jnp.int32)
             .reshape(BPC, BPC, NW, 2, PPC)
             .transpose(2, 0, 3, 1, 4)
             .reshape(NW, NCHUNK, RPC))
    return _emb(ids_r, tok_table, pos_table, type_table, gamma, beta)
